# baseline (speedup 1.0000x reference)
import functools

import jax
import jax.numpy as jnp
from jax import lax
from jax.experimental import pallas as pl
from jax.experimental.pallas import tpu as pltpu
from jax.experimental.pallas import tpu_sc as plsc

N_NODES = 1000000
N_RELATIONS = 1000
EMBED_DIM = 32
BATCH = 16384

_INFO = plsc.get_sparse_core_info()
_NC = _INFO.num_cores
_NS = _INFO.num_subcores
_NW = _NC * _NS
_L = _INFO.num_lanes

_B_PER_W = BATCH // _NW
_CHUNK = 256
_N_CHUNKS = _B_PER_W // _CHUNK
_GROUPS = _CHUNK // _L


def _body(hs_hbm, hp_hbm, ts_hbm, tp_hbm, rels_hbm,
          nr_hbm, ni_hbm, rTr_hbm, rTi_hbm,
          out_hbm,
          hs_v, hp_v, ts_v, tp_v, r_idx,
          srT, siT, trT, tiT,
          relr_v, reli_v,
          out_v, sem, rsem):
    wid = lax.axis_index("s") * _NC + lax.axis_index("c")
    base = wid * _B_PER_W

    cr = pltpu.async_copy(rTr_hbm, relr_v, rsem)
    ci = pltpu.async_copy(rTi_hbm, reli_v, rsem)

    pltpu.sync_copy(hs_hbm.at[pl.ds(base, _B_PER_W)], hs_v)
    pltpu.sync_copy(hp_hbm.at[pl.ds(base, _B_PER_W)], hp_v)
    pltpu.sync_copy(ts_hbm.at[pl.ds(base, _B_PER_W)], ts_v)
    pltpu.sync_copy(tp_hbm.at[pl.ds(base, _B_PER_W)], tp_v)
    pltpu.sync_copy(rels_hbm.at[pl.ds(base, _B_PER_W)], r_idx)

    cr.wait()
    ci.wait()

    for chunk in range(_N_CHUNKS):
        off = chunk * _CHUNK

        def issue(g, carry):
            goff = g * _L
            h_ids = hs_v[pl.ds(off + goff, _L)]
            t_ids = ts_v[pl.ds(off + goff, _L)]
            for c in range(EMBED_DIM):
                dpos = pl.ds(c * _CHUNK + goff, _L)
                pltpu.async_copy(nr_hbm.at[c].at[h_ids], srT.at[dpos], sem)
                pltpu.async_copy(ni_hbm.at[c].at[h_ids], siT.at[dpos], sem)
                pltpu.async_copy(nr_hbm.at[c].at[t_ids], trT.at[dpos], sem)
                pltpu.async_copy(ni_hbm.at[c].at[t_ids], tiT.at[dpos], sem)
            return carry

        lax.fori_loop(0, _GROUPS, issue, 0)

        nelems = EMBED_DIM * _CHUNK
        dummy = nr_hbm.at[0].at[pl.ds(0, nelems)]
        pltpu.make_async_copy(dummy, srT, sem).wait()
        pltpu.make_async_copy(dummy, siT, sem).wait()
        pltpu.make_async_copy(dummy, trT, sem).wait()
        pltpu.make_async_copy(dummy, tiT, sem).wait()

        def compute(g, carry):
            goff = g * _L
            rel_ids = r_idx[pl.ds(off + goff, _L)]
            h_sp = hp_v[pl.ds(off + goff, _L)]
            t_sp = tp_v[pl.ds(off + goff, _L)]
            phi = jnp.zeros((_L,), jnp.float32)
            for c in range(EMBED_DIM):
                cbase = c * _CHUNK
                flat_ids = rel_ids + (c * N_RELATIONS)
                sr_c = plsc.load_gather(srT, [h_sp + cbase])
                si_c = plsc.load_gather(siT, [h_sp + cbase])
                tr_c = plsc.load_gather(trT, [t_sp + cbase])
                ti_c = plsc.load_gather(tiT, [t_sp + cbase])
                rr_c = plsc.load_gather(relr_v, [flat_ids])
                ri_c = plsc.load_gather(reli_v, [flat_ids])
                phi = phi + rr_c * (sr_c * tr_c + si_c * ti_c)
                phi = phi + ri_c * (sr_c * ti_c - si_c * tr_c)
            out_v[pl.ds(off + goff, _L)] = phi
            return carry

        lax.fori_loop(0, _GROUPS, compute, 0)

    pltpu.sync_copy(out_v, out_hbm.at[pl.ds(base, _B_PER_W)])


@jax.jit
def kernel(heads, rels, tails, node_r, node_i, rel_r, rel_i):
    mesh = plsc.VectorSubcoreMesh(core_axis_name="c", subcore_axis_name="s")
    f = functools.partial(
        pl.kernel,
        out_type=jax.ShapeDtypeStruct((BATCH,), jnp.float32),
        mesh=mesh,
        compiler_params=pltpu.CompilerParams(
            use_tc_tiling_on_sc=False, needs_layout_passes=False),
        scratch_types=[
            pltpu.VMEM((_B_PER_W,), jnp.int32),
            pltpu.VMEM((_B_PER_W,), jnp.int32),
            pltpu.VMEM((_B_PER_W,), jnp.int32),
            pltpu.VMEM((_B_PER_W,), jnp.int32),
            pltpu.VMEM((_B_PER_W,), jnp.int32),
            pltpu.VMEM((EMBED_DIM * _CHUNK,), jnp.float32),
            pltpu.VMEM((EMBED_DIM * _CHUNK,), jnp.float32),
            pltpu.VMEM((EMBED_DIM * _CHUNK,), jnp.float32),
            pltpu.VMEM((EMBED_DIM * _CHUNK,), jnp.float32),
            pltpu.VMEM((EMBED_DIM * N_RELATIONS,), jnp.float32),
            pltpu.VMEM((EMBED_DIM * N_RELATIONS,), jnp.float32),
            pltpu.VMEM((_B_PER_W,), jnp.float32),
            pltpu.SemaphoreType.DMA,
            pltpu.SemaphoreType.DMA,
        ],
    )(_body)

    def chunk_sort(ids):
        ids2 = ids.reshape(-1, _CHUNK)
        order = jnp.argsort(ids2, axis=1)
        inv = jnp.argsort(order, axis=1).astype(jnp.int32)
        srt = jnp.take_along_axis(ids2, order, axis=1)
        return srt.reshape(-1), inv.reshape(-1)

    hs, hp = chunk_sort(heads)
    ts, tp = chunk_sort(tails)
    return f(hs, hp, ts, tp, rels, node_r.T, node_i.T,
             rel_r.T.reshape(EMBED_DIM * N_RELATIONS),
             rel_i.T.reshape(EMBED_DIM * N_RELATIONS))


# speedup vs baseline: 5.7766x; 5.7766x over previous
import functools

import jax
import jax.numpy as jnp
from jax import lax
from jax.experimental import pallas as pl
from jax.experimental.pallas import tpu as pltpu
from jax.experimental.pallas import tpu_sc as plsc

N_NODES = 1000000
N_RELATIONS = 1000
EMBED_DIM = 32
BATCH = 16384

_INFO = plsc.get_sparse_core_info()
_NC = _INFO.num_cores
_NS = _INFO.num_subcores
_NW = _NC * _NS
_L = _INFO.num_lanes

_B_PER_W = BATCH // _NW
_GROUPS = _B_PER_W // _L

_GATHER_DNUMS = lax.GatherDimensionNumbers(
    offset_dims=(), collapsed_slice_dims=(0,), start_index_map=(0,))


def _lane_perm(x, p):
    return lax.gather(x, p[:, None], dimension_numbers=_GATHER_DNUMS,
                      slice_sizes=(1,),
                      mode=lax.GatherScatterMode.PROMISE_IN_BOUNDS)


def _body(heads_hbm, rels_hbm, tails_hbm,
          node_r_hbm, node_i_hbm, rel_r_hbm, rel_i_hbm,
          out_hbm,
          h_idx, r_idx, t_idx,
          sr, si, rr, ri, tr, ti,
          out_v, sem):
    wid = lax.axis_index("s") * _NC + lax.axis_index("c")
    base = wid * _B_PER_W

    pltpu.sync_copy(heads_hbm.at[pl.ds(base, _B_PER_W)], h_idx)
    pltpu.sync_copy(rels_hbm.at[pl.ds(base, _B_PER_W)], r_idx)
    pltpu.sync_copy(tails_hbm.at[pl.ds(base, _B_PER_W)], t_idx)

    c1 = pltpu.async_copy(node_r_hbm.at[h_idx], sr, sem)
    c2 = pltpu.async_copy(node_i_hbm.at[h_idx], si, sem)
    c3 = pltpu.async_copy(rel_r_hbm.at[r_idx], rr, sem)
    c4 = pltpu.async_copy(rel_i_hbm.at[r_idx], ri, sem)
    c5 = pltpu.async_copy(node_r_hbm.at[t_idx], tr, sem)
    c6 = pltpu.async_copy(node_i_hbm.at[t_idx], ti, sem)
    c1.wait()
    c2.wait()
    c3.wait()
    c4.wait()
    c5.wait()
    c6.wait()

    lane = lax.iota(jnp.int32, _L)
    perms = [lane ^ s for s in (8, 4, 2, 1)]

    def group(g, carry):
        acc_out = jnp.zeros((_L,), jnp.float32)
        for j in range(_L):
            b = g * _L + j
            sr0 = sr[b, pl.ds(0, _L)]
            sr1 = sr[b, pl.ds(_L, _L)]
            si0 = si[b, pl.ds(0, _L)]
            si1 = si[b, pl.ds(_L, _L)]
            rr0 = rr[b, pl.ds(0, _L)]
            rr1 = rr[b, pl.ds(_L, _L)]
            ri0 = ri[b, pl.ds(0, _L)]
            ri1 = ri[b, pl.ds(_L, _L)]
            tr0 = tr[b, pl.ds(0, _L)]
            tr1 = tr[b, pl.ds(_L, _L)]
            ti0 = ti[b, pl.ds(0, _L)]
            ti1 = ti[b, pl.ds(_L, _L)]
            a0 = rr0 * sr0 - ri0 * si0
            b0 = rr0 * si0 + ri0 * sr0
            a1 = rr1 * sr1 - ri1 * si1
            b1 = rr1 * si1 + ri1 * sr1
            acc = (a0 * tr0 + b0 * ti0) + (a1 * tr1 + b1 * ti1)
            for p in perms:
                acc = acc + _lane_perm(acc, p)
            acc_out = jnp.where(lane == j, acc, acc_out)
        out_v[pl.ds(g * _L, _L)] = acc_out
        return carry

    lax.fori_loop(0, _GROUPS, group, 0)

    pltpu.sync_copy(out_v, out_hbm.at[pl.ds(base, _B_PER_W)])


@jax.jit
def kernel(heads, rels, tails, node_r, node_i, rel_r, rel_i):
    mesh = plsc.VectorSubcoreMesh(core_axis_name="c", subcore_axis_name="s")
    f = functools.partial(
        pl.kernel,
        out_type=jax.ShapeDtypeStruct((BATCH,), jnp.float32),
        mesh=mesh,
        compiler_params=pltpu.CompilerParams(use_tc_tiling_on_sc=False),
        scratch_types=[
            pltpu.VMEM((_B_PER_W,), jnp.int32),
            pltpu.VMEM((_B_PER_W,), jnp.int32),
            pltpu.VMEM((_B_PER_W,), jnp.int32),
            pltpu.VMEM((_B_PER_W, EMBED_DIM), jnp.float32),
            pltpu.VMEM((_B_PER_W, EMBED_DIM), jnp.float32),
            pltpu.VMEM((_B_PER_W, EMBED_DIM), jnp.float32),
            pltpu.VMEM((_B_PER_W, EMBED_DIM), jnp.float32),
            pltpu.VMEM((_B_PER_W, EMBED_DIM), jnp.float32),
            pltpu.VMEM((_B_PER_W, EMBED_DIM), jnp.float32),
            pltpu.VMEM((_B_PER_W,), jnp.float32),
            pltpu.SemaphoreType.DMA,
        ],
    )(_body)
    return f(heads, rels, tails, node_r, node_i, rel_r, rel_i)
